# baseline (speedup 1.0000x reference)
import functools

import jax
import jax.numpy as jnp
from jax import lax
from jax.experimental import pallas as pl
from jax.experimental.pallas import tpu as pltpu
from jax.experimental.pallas import tpu_sc as plsc

B = 16384
K = 32
NC = 2
NS = 16
NW = NC * NS
BPW = B // NW
CHUNK = 256
QUAD = 4
NPAR = 3


def _sc_gather(uidx, iidx, Wt, Ht, ubf, ibf):
    mesh = plsc.VectorSubcoreMesh(core_axis_name="c", subcore_axis_name="s")

    @functools.partial(
        pl.kernel,
        out_type=(
            jax.ShapeDtypeStruct((K, B), jnp.float32),
            jax.ShapeDtypeStruct((K, B), jnp.float32),
            jax.ShapeDtypeStruct((B,), jnp.float32),
            jax.ShapeDtypeStruct((B,), jnp.float32),
        ),
        mesh=mesh,
        scratch_types=[
            pltpu.VMEM((BPW + 16,), jnp.int32),
            pltpu.VMEM((BPW + 16,), jnp.int32),
            [pltpu.VMEM((K, 128), jnp.float32) for _ in range(NPAR * QUAD)],
            [pltpu.VMEM((K, 128), jnp.float32) for _ in range(NPAR * QUAD)],
            pltpu.VMEM((K, CHUNK), jnp.float32),
            pltpu.VMEM((K, CHUNK), jnp.float32),
            pltpu.VMEM((BPW,), jnp.float32),
            pltpu.VMEM((BPW,), jnp.float32),
            [pltpu.SemaphoreType.DMA for _ in range(2 * NPAR * QUAD)],
            pltpu.SemaphoreType.DMA,
        ],
        compiler_params=pltpu.CompilerParams(needs_layout_passes=False),
    )
    def gather_kernel(uidx_hbm, iidx_hbm, wt_hbm, ht_hbm, ub_hbm, ib_hbm,
                      ut_hbm, vt_hbm, ubo_hbm, ibo_hbm,
                      uidx_v, iidx_v, wslots, hslots, uo_v, vo_v,
                      ubg_v, ibg_v, sems, bsem):
        wid = lax.axis_index("s") * NC + lax.axis_index("c")
        base = wid * BPW
        pltpu.sync_copy(uidx_hbm.at[pl.ds(base, BPW)], uidx_v.at[pl.ds(0, BPW)])
        pltpu.sync_copy(iidx_hbm.at[pl.ds(base, BPW)], iidx_v.at[pl.ds(0, BPW)])
        bcps = (
            pltpu.async_copy(ub_hbm.at[uidx_v.at[pl.ds(0, BPW)]], ubg_v, bsem),
            pltpu.async_copy(ib_hbm.at[iidx_v.at[pl.ds(0, BPW)]], ibg_v, bsem),
        )

        klo = lax.iota(jnp.int32, 16)
        khi = klo + 16

        def fire_one(tbl_hbm, idx_vec, lane, slot, sem):
            col = idx_vec[lane]
            off = pl.multiple_of((col >> 7) << 7, 128)
            return pltpu.async_copy(tbl_hbm.at[:, pl.ds(off, 128)], slot, sem)

        def fire_batch(sbase, par):
            uvec = uidx_v[pl.ds(sbase, 16)]
            ivec = iidx_v[pl.ds(sbase, 16)]
            for s in range(QUAD):
                fire_one(wt_hbm, uvec, s, wslots[par * QUAD + s],
                         sems[par * QUAD + s])
                fire_one(ht_hbm, ivec, s, hslots[par * QUAD + s],
                         sems[NPAR * QUAD + par * QUAD + s])

        def extract_one(idx_vec, lane, slot, out_ref, jc):
            col = idx_vec[lane]
            c = jnp.broadcast_to(col & 127, (16,))
            jcv = jnp.broadcast_to(jc, (16,))
            lo = plsc.load_gather(slot, [klo, c])
            hi = plsc.load_gather(slot, [khi, c])
            plsc.store_scatter(out_ref, [klo, jcv], lo)
            plsc.store_scatter(out_ref, [khi, jcv], hi)

        def drain_extract_batch(sbase, par, jcbase):
            uvec = uidx_v[pl.ds(sbase, 16)]
            ivec = iidx_v[pl.ds(sbase, 16)]
            for s in range(QUAD):
                pltpu.make_async_copy(
                    wt_hbm.at[:, pl.ds(0, 128)], wslots[par * QUAD + s],
                    sems[par * QUAD + s]).wait()
                extract_one(uvec, s, wslots[par * QUAD + s], uo_v, jcbase + s)
                pltpu.make_async_copy(
                    ht_hbm.at[:, pl.ds(0, 128)], hslots[par * QUAD + s],
                    sems[NPAR * QUAD + par * QUAD + s]).wait()
                extract_one(ivec, s, hslots[par * QUAD + s], vo_v, jcbase + s)

        NB = CHUNK // QUAD
        STEADY = ((NB - NPAR + 1) // NPAR) * NPAR
        for ch in range(BPW // CHUNK):
            chbase = ch * CHUNK
            fire_batch(chbase, 0)
            fire_batch(chbase + QUAD, 1)

            def step(k, jc):
                par = k % NPAR
                fire_batch(chbase + jc + (NPAR - 1) * QUAD, (k + 2) % NPAR)
                drain_extract_batch(chbase + jc, par, jc)

            def body(p, _):
                jc0 = p * NPAR * QUAD
                for t in range(NPAR):
                    step(t, jc0 + t * QUAD)
                return 0

            lax.fori_loop(0, STEADY // NPAR, body, 0, unroll=False)
            for k in range(STEADY, NB):
                par = k % NPAR
                if k + NPAR - 1 < NB:
                    fire_batch(chbase + (k + NPAR - 1) * QUAD,
                               (k + NPAR - 1) % NPAR)
                drain_extract_batch(chbase + k * QUAD, par, k * QUAD)
            pltpu.sync_copy(uo_v, ut_hbm.at[:, pl.ds(base + chbase, CHUNK)])
            pltpu.sync_copy(vo_v, vt_hbm.at[:, pl.ds(base + chbase, CHUNK)])

        for cp in bcps:
            cp.wait()
        pltpu.sync_copy(ubg_v, ubo_hbm.at[pl.ds(base, BPW)])
        pltpu.sync_copy(ibg_v, ibo_hbm.at[pl.ds(base, BPW)])

    return gather_kernel(uidx, iidx, Wt, Ht, ubf, ibf)


def _flat_body(u_ref, i_ref, uo_ref, io_ref):
    uo_ref[...] = u_ref[0, :]
    io_ref[...] = i_ref[0, :]


def _tc_flatten(ubT, ibT):
    n = ubT.shape[1]
    blk = 131072
    grid = (pl.cdiv(n, blk),)
    return pl.pallas_call(
        _flat_body,
        grid=grid,
        in_specs=[
            pl.BlockSpec((1, blk), lambda i: (0, i)),
            pl.BlockSpec((1, blk), lambda i: (0, i)),
        ],
        out_specs=[
            pl.BlockSpec((blk,), lambda i: (i,)),
            pl.BlockSpec((blk,), lambda i: (i,)),
        ],
        out_shape=[
            jax.ShapeDtypeStruct((n,), jnp.float32),
            jax.ShapeDtypeStruct((n,), jnp.float32),
        ],
    )(ubT, ibT)


def _mlp_body(ut_ref, vt_ref, ub_ref, ib_ref, w1a_ref, w1b_ref, b1_ref,
              w2_ref, o_ref):
    h = (
        jnp.dot(w1a_ref[...], ut_ref[...], preferred_element_type=jnp.float32)
        + jnp.dot(w1b_ref[...], vt_ref[...], preferred_element_type=jnp.float32)
        + b1_ref[...]
    )
    h = jnp.maximum(h, 0.0)
    acc = jnp.sum(h * w2_ref[...], axis=0, keepdims=True)
    o_ref[...] = acc + ub_ref[...] + ib_ref[...]


def _tc_mlp(Ut, Vt, ubg, ibg, w1a, w1b, b1c, w2c):
    blk = 2048
    grid = (B // blk,)
    return pl.pallas_call(
        _mlp_body,
        grid=grid,
        in_specs=[
            pl.BlockSpec((K, blk), lambda i: (0, i)),
            pl.BlockSpec((K, blk), lambda i: (0, i)),
            pl.BlockSpec((1, blk), lambda i: (0, i)),
            pl.BlockSpec((1, blk), lambda i: (0, i)),
            pl.BlockSpec((K, K), lambda i: (0, 0)),
            pl.BlockSpec((K, K), lambda i: (0, 0)),
            pl.BlockSpec((K, 1), lambda i: (0, 0)),
            pl.BlockSpec((K, 1), lambda i: (0, 0)),
        ],
        out_specs=pl.BlockSpec((1, blk), lambda i: (0, i)),
        out_shape=jax.ShapeDtypeStruct((1, B), jnp.float32),
    )(Ut, Vt, ubg, ibg, w1a, w1b, b1c, w2c)


def kernel(x, W, H, W_pre, H_pre, W_eps, H_eps, W1, b1, W2, ub, ib):
    uidx = x[:, 0]
    iidx = x[:, 1]
    ubf, ibf = _tc_flatten(ub.T, ib.T)
    Ut, Vt, ubg, ibg = _sc_gather(uidx, iidx, W.T, H.T, ubf, ibf)
    w1a = W1[:, :K]
    w1b = W1[:, K:]
    outT = _tc_mlp(Ut, Vt, ubg.reshape(1, B), ibg.reshape(1, B),
                   w1a, w1b, b1.reshape(K, 1), W2.reshape(K, 1))
    return outT.reshape(B, 1)


# speedup vs baseline: 1.0688x; 1.0688x over previous
import functools

import jax
import jax.numpy as jnp
from jax import lax
from jax.experimental import pallas as pl
from jax.experimental.pallas import tpu as pltpu
from jax.experimental.pallas import tpu_sc as plsc

B = 16384
K = 32
NC = 2
NS = 16
NW = NC * NS
BPW = B // NW
CHUNK = 256
QUAD = 4
NPAR = 3


def _sc_gather(uidx, iidx, Wt, Ht, ubf, ibf):
    mesh = plsc.VectorSubcoreMesh(core_axis_name="c", subcore_axis_name="s")

    @functools.partial(
        pl.kernel,
        out_type=(
            jax.ShapeDtypeStruct((K, B), jnp.float32),
            jax.ShapeDtypeStruct((K, B), jnp.float32),
            jax.ShapeDtypeStruct((B,), jnp.float32),
            jax.ShapeDtypeStruct((B,), jnp.float32),
        ),
        mesh=mesh,
        scratch_types=[
            pltpu.VMEM((BPW + 16,), jnp.int32),
            pltpu.VMEM((BPW + 16,), jnp.int32),
            [pltpu.VMEM((K, 128), jnp.float32) for _ in range(NPAR * QUAD)],
            [pltpu.VMEM((K, 128), jnp.float32) for _ in range(NPAR * QUAD)],
            pltpu.VMEM((K, CHUNK), jnp.float32),
            pltpu.VMEM((K, CHUNK), jnp.float32),
            pltpu.VMEM((BPW,), jnp.float32),
            pltpu.VMEM((BPW,), jnp.float32),
            [pltpu.SemaphoreType.DMA for _ in range(2 * NPAR)],
            pltpu.SemaphoreType.DMA,
        ],
        compiler_params=pltpu.CompilerParams(needs_layout_passes=False),
    )
    def gather_kernel(uidx_hbm, iidx_hbm, wt_hbm, ht_hbm, ub_hbm, ib_hbm,
                      ut_hbm, vt_hbm, ubo_hbm, ibo_hbm,
                      uidx_v, iidx_v, wslots, hslots, uo_v, vo_v,
                      ubg_v, ibg_v, sems, bsem):
        wid = lax.axis_index("s") * NC + lax.axis_index("c")
        base = wid * BPW
        pltpu.sync_copy(uidx_hbm.at[pl.ds(base, BPW)], uidx_v.at[pl.ds(0, BPW)])
        pltpu.sync_copy(iidx_hbm.at[pl.ds(base, BPW)], iidx_v.at[pl.ds(0, BPW)])
        bcps = (
            pltpu.async_copy(ub_hbm.at[uidx_v.at[pl.ds(0, BPW)]], ubg_v, bsem),
            pltpu.async_copy(ib_hbm.at[iidx_v.at[pl.ds(0, BPW)]], ibg_v, bsem),
        )

        klo = lax.iota(jnp.int32, 16)
        khi = klo + 16

        def fire_one(tbl_hbm, idx_vec, lane, slot, sem):
            col = idx_vec[lane]
            off = pl.multiple_of((col >> 7) << 7, 128)
            return pltpu.async_copy(tbl_hbm.at[:, pl.ds(off, 128)], slot, sem)

        def fire_batch(sbase, par):
            uvec = uidx_v[pl.ds(sbase, 16)]
            ivec = iidx_v[pl.ds(sbase, 16)]
            for s in range(QUAD):
                fire_one(wt_hbm, uvec, s, wslots[par * QUAD + s], sems[par])
                fire_one(ht_hbm, ivec, s, hslots[par * QUAD + s],
                         sems[NPAR + par])

        def extract_one(idx_vec, lane, slot, out_ref, jc):
            col = idx_vec[lane]
            c = jnp.broadcast_to(col & 127, (16,))
            jcv = jnp.broadcast_to(jc, (16,))
            lo = plsc.load_gather(slot, [klo, c])
            hi = plsc.load_gather(slot, [khi, c])
            plsc.store_scatter(out_ref, [klo, jcv], lo)
            plsc.store_scatter(out_ref, [khi, jcv], hi)

        def drain_extract_batch(sbase, par, jcbase):
            uvec = uidx_v[pl.ds(sbase, 16)]
            ivec = iidx_v[pl.ds(sbase, 16)]
            for s in range(QUAD):
                pltpu.make_async_copy(
                    wt_hbm.at[:, pl.ds(0, 128)], wslots[par * QUAD + s],
                    sems[par]).wait()
                pltpu.make_async_copy(
                    ht_hbm.at[:, pl.ds(0, 128)], hslots[par * QUAD + s],
                    sems[NPAR + par]).wait()
            for s in range(QUAD):
                extract_one(uvec, s, wslots[par * QUAD + s], uo_v, jcbase + s)
                extract_one(ivec, s, hslots[par * QUAD + s], vo_v, jcbase + s)

        NB = CHUNK // QUAD
        STEADY = ((NB - NPAR + 1) // NPAR) * NPAR
        for ch in range(BPW // CHUNK):
            chbase = ch * CHUNK
            fire_batch(chbase, 0)
            fire_batch(chbase + QUAD, 1)

            def step(k, jc):
                par = k % NPAR
                fire_batch(chbase + jc + (NPAR - 1) * QUAD, (k + 2) % NPAR)
                drain_extract_batch(chbase + jc, par, jc)

            def body(p, _):
                jc0 = p * NPAR * QUAD
                for t in range(NPAR):
                    step(t, jc0 + t * QUAD)
                return 0

            lax.fori_loop(0, STEADY // NPAR, body, 0, unroll=False)
            for k in range(STEADY, NB):
                par = k % NPAR
                if k + NPAR - 1 < NB:
                    fire_batch(chbase + (k + NPAR - 1) * QUAD,
                               (k + NPAR - 1) % NPAR)
                drain_extract_batch(chbase + k * QUAD, par, k * QUAD)
            pltpu.sync_copy(uo_v, ut_hbm.at[:, pl.ds(base + chbase, CHUNK)])
            pltpu.sync_copy(vo_v, vt_hbm.at[:, pl.ds(base + chbase, CHUNK)])

        for cp in bcps:
            cp.wait()
        pltpu.sync_copy(ubg_v, ubo_hbm.at[pl.ds(base, BPW)])
        pltpu.sync_copy(ibg_v, ibo_hbm.at[pl.ds(base, BPW)])

    return gather_kernel(uidx, iidx, Wt, Ht, ubf, ibf)


def _flat_body(u_ref, i_ref, uo_ref, io_ref):
    uo_ref[...] = u_ref[0, :]
    io_ref[...] = i_ref[0, :]


def _tc_flatten(ubT, ibT):
    n = ubT.shape[1]
    blk = 131072
    grid = (pl.cdiv(n, blk),)
    return pl.pallas_call(
        _flat_body,
        grid=grid,
        in_specs=[
            pl.BlockSpec((1, blk), lambda i: (0, i)),
            pl.BlockSpec((1, blk), lambda i: (0, i)),
        ],
        out_specs=[
            pl.BlockSpec((blk,), lambda i: (i,)),
            pl.BlockSpec((blk,), lambda i: (i,)),
        ],
        out_shape=[
            jax.ShapeDtypeStruct((n,), jnp.float32),
            jax.ShapeDtypeStruct((n,), jnp.float32),
        ],
    )(ubT, ibT)


def _mlp_body(ut_ref, vt_ref, ub_ref, ib_ref, w1a_ref, w1b_ref, b1_ref,
              w2_ref, o_ref):
    h = (
        jnp.dot(w1a_ref[...], ut_ref[...], preferred_element_type=jnp.float32)
        + jnp.dot(w1b_ref[...], vt_ref[...], preferred_element_type=jnp.float32)
        + b1_ref[...]
    )
    h = jnp.maximum(h, 0.0)
    acc = jnp.sum(h * w2_ref[...], axis=0, keepdims=True)
    o_ref[...] = acc + ub_ref[...] + ib_ref[...]


def _tc_mlp(Ut, Vt, ubg, ibg, w1a, w1b, b1c, w2c):
    blk = 2048
    grid = (B // blk,)
    return pl.pallas_call(
        _mlp_body,
        grid=grid,
        in_specs=[
            pl.BlockSpec((K, blk), lambda i: (0, i)),
            pl.BlockSpec((K, blk), lambda i: (0, i)),
            pl.BlockSpec((1, blk), lambda i: (0, i)),
            pl.BlockSpec((1, blk), lambda i: (0, i)),
            pl.BlockSpec((K, K), lambda i: (0, 0)),
            pl.BlockSpec((K, K), lambda i: (0, 0)),
            pl.BlockSpec((K, 1), lambda i: (0, 0)),
            pl.BlockSpec((K, 1), lambda i: (0, 0)),
        ],
        out_specs=pl.BlockSpec((1, blk), lambda i: (0, i)),
        out_shape=jax.ShapeDtypeStruct((1, B), jnp.float32),
    )(Ut, Vt, ubg, ibg, w1a, w1b, b1c, w2c)


def kernel(x, W, H, W_pre, H_pre, W_eps, H_eps, W1, b1, W2, ub, ib):
    uidx = x[:, 0]
    iidx = x[:, 1]
    ubf, ibf = _tc_flatten(ub.T, ib.T)
    Ut, Vt, ubg, ibg = _sc_gather(uidx, iidx, W.T, H.T, ubf, ibf)
    w1a = W1[:, :K]
    w1b = W1[:, K:]
    outT = _tc_mlp(Ut, Vt, ubg.reshape(1, B), ibg.reshape(1, B),
                   w1a, w1b, b1.reshape(K, 1), W2.reshape(K, 1))
    return outT.reshape(B, 1)
